# baseline (speedup 1.0000x reference)
import dataclasses
import functools

import jax
import jax.numpy as jnp
from jax import lax
from jax.experimental import pallas as pl
from jax.experimental.pallas import tpu as pltpu
from jax.experimental.pallas import tpu_sc as plsc

N = 10000
E = 320000
D = 128
HD = D // 2
NC = 2
NS = 16
NW = NC * NS
CH = 128
KCH = 80
NBUF = 5
KT = 160
NCHUNK = NS * KT
EPAD = NCHUNK * CH
ACC_ROWS = 10240
RPS = ACC_ROWS // NS

_vec_mesh = plsc.VectorSubcoreMesh(core_axis_name="c", subcore_axis_name="s")
_vec_mesh1 = plsc.VectorSubcoreMesh(
    core_axis_name="c", subcore_axis_name="s", num_cores=1, num_subcores=NS)

_sc_params = pltpu.CompilerParams()
if "needs_layout_passes" in pltpu.CompilerParams.__dataclass_fields__:
    _sc_params = dataclasses.replace(_sc_params, needs_layout_passes=False)
_sc_agg_params = dataclasses.replace(_sc_params, use_tc_tiling_on_sc=False)


def _hist_body(dst_hbm, out_hbm, dst_v, hist_v):
    cid = lax.axis_index("c")
    sid = lax.axis_index("s")
    wid = sid * NC + cid
    pltpu.sync_copy(dst_hbm.at[wid], dst_v)
    zero16 = jnp.zeros((16,), jnp.float32)
    ones16 = jnp.ones((16,), jnp.float32)

    @pl.loop(0, ACC_ROWS // 16)
    def _zero(r):
        hist_v[r, :] = zero16

    @pl.loop(0, KCH)
    def _chunk(j):
        @pl.loop(0, CH // 16)
        def _grp(c):
            idx = dst_v[j, pl.ds(c * 16, 16)]
            row = lax.shift_right_logical(idx, 4)
            col = lax.bitwise_and(idx, 15)
            plsc.addupdate_scatter(hist_v, [row, col], ones16)

    pltpu.sync_copy(hist_v, out_hbm.at[wid])


def _sc_hist(dst3):
    k = pl.kernel(
        _hist_body,
        out_type=jax.ShapeDtypeStruct((NW, ACC_ROWS // 16, 16), jnp.float32),
        mesh=_vec_mesh,
        scratch_types=[
            pltpu.VMEM((KCH, CH), jnp.int32),
            pltpu.VMEM((ACC_ROWS // 16, 16), jnp.float32),
        ],
        compiler_params=_sc_params,
    )
    return k(dst3)


def _agg_body(g_hbm, src_hbm, dst_hbm, out_hbm, src_v, dst_v, *rest):
    bufs = rest[:NBUF]
    acc_sh = rest[NBUF]
    pl.run_scoped(
        functools.partial(_agg_inner, g_hbm, src_hbm, dst_hbm, out_hbm,
                          src_v, dst_v, bufs, acc_sh),
        gsems=pltpu.SemaphoreType.DMA((NBUF,)),
        ssems=pltpu.SemaphoreType.DMA((NBUF,)),
    )


def _agg_ring(g_hbm, sv, dv, bufs, acc_sh, gsem, ssem, K):
    for b in range(NBUF):
        pltpu.async_copy(g_hbm.at[sv.at[b]], bufs[b], gsem[b])

    @pl.loop(0, K // NBUF - 1)
    def _grp(t):
        j0 = t * NBUF
        for b in range(NBUF):
            pltpu.make_async_copy(
                g_hbm.at[sv.at[j0 + b]], bufs[b], gsem[b]).wait()
            pltpu.async_copy(
                bufs[b], acc_sh.at[dv.at[j0 + b]], ssem[b], add=True)
        for b in range(NBUF):
            pltpu.make_async_copy(
                bufs[b], acc_sh.at[dv.at[j0 + b]], ssem[b]).wait()
            pltpu.async_copy(
                g_hbm.at[sv.at[j0 + NBUF + b]], bufs[b], gsem[b])

    jlast = K - NBUF
    for b in range(NBUF):
        pltpu.make_async_copy(
            g_hbm.at[sv.at[jlast + b]], bufs[b], gsem[b]).wait()
        pltpu.async_copy(
            bufs[b], acc_sh.at[dv.at[jlast + b]], ssem[b], add=True)
    for b in range(NBUF):
        pltpu.make_async_copy(
            bufs[b], acc_sh.at[dv.at[jlast + b]], ssem[b]).wait()


def _agg_inner(g_hbm, src_hbm, dst_hbm, out_hbm, src_v, dst_v,
               bufs, acc_sh, gsems, ssems):
    sid = lax.axis_index("s")
    gsem = [gsems.at[i] for i in range(NBUF)]
    ssem = [ssems.at[i] for i in range(NBUF)]
    zero16 = jnp.zeros((16,), jnp.float32)
    one16 = jnp.full((16,), 1, jnp.int32)

    pltpu.sync_copy(src_hbm.at[pl.ds(sid * KT, KT)], src_v)
    pltpu.sync_copy(dst_hbm.at[pl.ds(sid * KT, KT)], dst_v)

    for p in range(2):
        if p == 1:
            @pl.loop(0, KT)
            def _incr(r):
                @pl.loop(0, CH // 16)
                def _incc(c):
                    src_v[r, pl.ds(c * 16, 16)] = (
                        src_v[r, pl.ds(c * 16, 16)] + one16)

        @pl.loop(0, CH)
        def _zr(r):
            @pl.loop(0, HD // 16)
            def _zc(c):
                bufs[0][r, pl.ds(c * 16, 16)] = zero16

        @pl.loop(0, RPS // CH)
        def _zacc(k):
            pltpu.sync_copy(bufs[0], acc_sh.at[pl.ds(sid * RPS + k * CH, CH)])

        plsc.subcore_barrier()
        _agg_ring(g_hbm, src_v, dst_v, bufs, acc_sh, gsem, ssem, KT)
        plsc.subcore_barrier()
        pltpu.sync_copy(
            acc_sh.at[pl.ds(sid * RPS, RPS)],
            out_hbm.at[p].at[pl.ds(sid * RPS, RPS)],
        )


def _sc_aggregate(g, src2c, dst2c):
    g2 = g.reshape(2 * N, HD)
    k = pl.kernel(
        _agg_body,
        out_type=jax.ShapeDtypeStruct((2, ACC_ROWS, HD), jnp.float32),
        mesh=_vec_mesh1,
        scratch_types=[
            pltpu.VMEM((KT, CH), jnp.int32),
            pltpu.VMEM((KT, CH), jnp.int32),
        ] + [pltpu.VMEM((CH, HD), jnp.float32)] * NBUF + [
            pltpu.VMEM_SHARED((ACC_ROWS, HD), jnp.float32),
        ],
        compiler_params=_sc_agg_params,
    )
    return k(g2, src2c, dst2c)


_RB = 2000


def _dinv_body(h_ref, o_ref):
    deg = jnp.sum(h_ref[...], axis=0) + 1.0
    o_ref[...] = 1.0 / jnp.sqrt(deg)


def _tc_dinv(hist4):
    return pl.pallas_call(
        _dinv_body,
        out_shape=jax.ShapeDtypeStruct((ACC_ROWS // D, D), jnp.float32),
    )(hist4)


def _mm_scale_body(x_ref, w_ref, dinv_ref, o_ref):
    h = lax.dot_general(
        x_ref[...], w_ref[...], (((1,), (0,)), ((), ())),
        precision=lax.Precision.HIGHEST,
    )
    o_ref[...] = h * dinv_ref[...]


def _tc_mm_scale(x, W, dinv):
    full = pl.BlockSpec((_RB, D), lambda i: (i, 0))
    col = pl.BlockSpec((_RB, 1), lambda i: (i, 0))
    return pl.pallas_call(
        _mm_scale_body,
        grid=(N // _RB,),
        in_specs=[full, pl.BlockSpec((D, D), lambda i: (0, 0)), col],
        out_specs=full,
        out_shape=jax.ShapeDtypeStruct((N, D), jnp.float32),
    )(x, W, dinv)


def _layer2_body(pL, pR, g1_ref, dinv_ref, b1_ref, w2_ref, o_ref):
    agg = jnp.concatenate([pL[...], pR[...]], axis=1)
    u = (agg + g1_ref[...]) * dinv_ref[...] + b1_ref[...]
    h = jnp.maximum(u, 0.0)
    o_ref[...] = lax.dot_general(
        h, w2_ref[...], (((1,), (0,)), ((), ())),
        precision=lax.Precision.HIGHEST,
    ) * dinv_ref[...]


def _tc_layer2(p, g1, dinv, b1r, W2):
    half = pl.BlockSpec((_RB, HD), lambda i: (i, 0))
    full = pl.BlockSpec((_RB, D), lambda i: (i, 0))
    col = pl.BlockSpec((_RB, 1), lambda i: (i, 0))
    return pl.pallas_call(
        _layer2_body,
        grid=(N // _RB,),
        in_specs=[half, half, full, col,
                  pl.BlockSpec((1, D), lambda i: (0, 0)),
                  pl.BlockSpec((D, D), lambda i: (0, 0))],
        out_specs=full,
        out_shape=jax.ShapeDtypeStruct((N, D), jnp.float32),
    )(p[0, :N], p[1, :N], g1, dinv, b1r, W2)


def _final_body(qL, qR, g2_ref, dinv_ref, b2_ref, o_ref):
    agg = jnp.concatenate([qL[...], qR[...]], axis=1)
    o_ref[...] = (agg + g2_ref[...]) * dinv_ref[...] + b2_ref[...]


def _tc_final(q, g2, dinv, b2r):
    half = pl.BlockSpec((_RB, HD), lambda i: (i, 0))
    full = pl.BlockSpec((_RB, D), lambda i: (i, 0))
    col = pl.BlockSpec((_RB, 1), lambda i: (i, 0))
    return pl.pallas_call(
        _final_body,
        grid=(N // _RB,),
        in_specs=[half, half, full, col,
                  pl.BlockSpec((1, D), lambda i: (0, 0))],
        out_specs=full,
        out_shape=jax.ShapeDtypeStruct((N, D), jnp.float32),
    )(q[0, :N], q[1, :N], g2, dinv, b2r)


def kernel(x, edge_index, W1, b1, W2, b2):
    ei = edge_index.astype(jnp.int32)
    pad = EPAD - E
    src = jnp.concatenate([ei[0], jnp.zeros((pad,), jnp.int32)])
    dump = N + (jnp.arange(pad, dtype=jnp.int32) % (ACC_ROWS - N))
    dst = jnp.concatenate([ei[1], dump])
    src2c = (2 * src).reshape(NCHUNK, CH)
    dst2c = dst.reshape(NCHUNK, CH)
    dst3 = dst.reshape(NW, KCH, CH)

    hist = _sc_hist(dst3)
    hist4 = hist.reshape(NW, ACC_ROWS // D, D)
    dinv = _tc_dinv(hist4).reshape(ACC_ROWS, 1)[:N]

    g1 = _tc_mm_scale(x, W1, dinv)
    p = _sc_aggregate(g1, src2c, dst2c)
    g2 = _tc_layer2(p, g1, dinv, b1.reshape(1, D), W2)
    q = _sc_aggregate(g2, src2c, dst2c)
    return _tc_final(q, g2, dinv, b2.reshape(1, D))


# speedup vs baseline: 1.6027x; 1.6027x over previous
import dataclasses

import jax
import jax.numpy as jnp
from jax import lax
from jax.experimental import pallas as pl
from jax.experimental.pallas import tpu as pltpu
from jax.experimental.pallas import tpu_sc as plsc

N = 10000
E = 320000
D = 128
NC = 2
NS = 16
NW = NC * NS
CH = 128
KCH = 79
EPAD = NW * KCH * CH
ACC_ROWS = 10240
RPS = ACC_ROWS // NS

_vec_mesh = plsc.VectorSubcoreMesh(core_axis_name="c", subcore_axis_name="s")

_sc_params = pltpu.CompilerParams()
if "needs_layout_passes" in pltpu.CompilerParams.__dataclass_fields__:
    _sc_params = dataclasses.replace(_sc_params, needs_layout_passes=False)


def _hist_body(dst_hbm, out_hbm, dst_v, hist_v):
    cid = lax.axis_index("c")
    sid = lax.axis_index("s")
    wid = sid * NC + cid
    pltpu.sync_copy(dst_hbm.at[wid], dst_v)
    zero16 = jnp.zeros((16,), jnp.float32)
    ones16 = jnp.ones((16,), jnp.float32)

    @pl.loop(0, ACC_ROWS // 16)
    def _zero(r):
        hist_v[r, :] = zero16

    @pl.loop(0, KCH)
    def _chunk(j):
        @pl.loop(0, CH // 16)
        def _grp(c):
            idx = dst_v[j, pl.ds(c * 16, 16)]
            row = lax.shift_right_logical(idx, 4)
            col = lax.bitwise_and(idx, 15)
            plsc.addupdate_scatter(hist_v, [row, col], ones16)

    pltpu.sync_copy(hist_v, out_hbm.at[wid])


def _sc_hist(dst3):
    k = pl.kernel(
        _hist_body,
        out_type=jax.ShapeDtypeStruct((NW, ACC_ROWS // 16, 16), jnp.float32),
        mesh=_vec_mesh,
        scratch_types=[
            pltpu.VMEM((KCH, CH), jnp.int32),
            pltpu.VMEM((ACC_ROWS // 16, 16), jnp.float32),
        ],
        compiler_params=_sc_params,
    )
    return k(dst3)


def _agg_body(g_hbm, src_hbm, dst_hbm, out_hbm, src_v, dst_v, buf, acc_sh):
    cid = lax.axis_index("c")
    sid = lax.axis_index("s")
    wid = sid * NC + cid
    zero16 = jnp.zeros((16,), jnp.float32)

    @pl.loop(0, CH)
    def _zr(r):
        @pl.loop(0, D // 16)
        def _zc(c):
            buf[r, pl.ds(c * 16, 16)] = zero16

    @pl.loop(0, RPS // CH)
    def _zacc(k):
        pltpu.sync_copy(buf, acc_sh.at[pl.ds(sid * RPS + k * CH, CH)])

    pltpu.sync_copy(src_hbm.at[wid], src_v)
    pltpu.sync_copy(dst_hbm.at[wid], dst_v)
    plsc.subcore_barrier()

    @pl.loop(0, KCH)
    def _edge_chunk(j):
        pltpu.sync_copy(g_hbm.at[src_v.at[j]], buf)
        pltpu.sync_copy(buf, acc_sh.at[dst_v.at[j]], add=True)

    plsc.subcore_barrier()
    pltpu.sync_copy(
        acc_sh.at[pl.ds(sid * RPS, RPS)],
        out_hbm.at[cid].at[pl.ds(sid * RPS, RPS)],
    )


def _sc_aggregate(g, src3, dst3):
    k = pl.kernel(
        _agg_body,
        out_type=jax.ShapeDtypeStruct((NC, ACC_ROWS, D), jnp.float32),
        mesh=_vec_mesh,
        scratch_types=[
            pltpu.VMEM((KCH, CH), jnp.int32),
            pltpu.VMEM((KCH, CH), jnp.int32),
            pltpu.VMEM((CH, D), jnp.float32),
            pltpu.VMEM_SHARED((ACC_ROWS, D), jnp.float32),
        ],
        compiler_params=_sc_params,
    )
    return k(g, src3, dst3)


_RB = 2000


def _dinv_body(h_ref, o_ref):
    deg = jnp.sum(h_ref[...], axis=0) + 1.0
    o_ref[...] = 1.0 / jnp.sqrt(deg)


def _tc_dinv(hist4):
    return pl.pallas_call(
        _dinv_body,
        out_shape=jax.ShapeDtypeStruct((ACC_ROWS // D, D), jnp.float32),
    )(hist4)


def _mm_scale_body(x_ref, w_ref, dinv_ref, o_ref):
    h = lax.dot_general(
        x_ref[...], w_ref[...], (((1,), (0,)), ((), ())),
        precision=lax.Precision.HIGHEST,
    )
    o_ref[...] = h * dinv_ref[...]


def _tc_mm_scale(x, W, dinv):
    full = pl.BlockSpec((_RB, D), lambda i: (i, 0))
    col = pl.BlockSpec((_RB, 1), lambda i: (i, 0))
    return pl.pallas_call(
        _mm_scale_body,
        grid=(N // _RB,),
        in_specs=[full, pl.BlockSpec((D, D), lambda i: (0, 0)), col],
        out_specs=full,
        out_shape=jax.ShapeDtypeStruct((N, D), jnp.float32),
    )(x, W, dinv)


def _layer2_body(p0_ref, p1_ref, g1_ref, dinv_ref, b1_ref, w2_ref, o_ref):
    u = (p0_ref[...] + p1_ref[...] + g1_ref[...]) * dinv_ref[...] + b1_ref[...]
    h = jnp.maximum(u, 0.0)
    o_ref[...] = lax.dot_general(
        h, w2_ref[...], (((1,), (0,)), ((), ())),
        precision=lax.Precision.HIGHEST,
    ) * dinv_ref[...]


def _tc_layer2(p0, p1, g1, dinv, b1r, W2):
    full = pl.BlockSpec((_RB, D), lambda i: (i, 0))
    col = pl.BlockSpec((_RB, 1), lambda i: (i, 0))
    return pl.pallas_call(
        _layer2_body,
        grid=(N // _RB,),
        in_specs=[full, full, full, col,
                  pl.BlockSpec((1, D), lambda i: (0, 0)),
                  pl.BlockSpec((D, D), lambda i: (0, 0))],
        out_specs=full,
        out_shape=jax.ShapeDtypeStruct((N, D), jnp.float32),
    )(p0, p1, g1, dinv, b1r, W2)


def _final_body(q0_ref, q1_ref, g2_ref, dinv_ref, b2_ref, o_ref):
    o_ref[...] = (
        (q0_ref[...] + q1_ref[...] + g2_ref[...]) * dinv_ref[...] + b2_ref[...]
    )


def _tc_final(q0, q1, g2, dinv, b2r):
    full = pl.BlockSpec((_RB, D), lambda i: (i, 0))
    col = pl.BlockSpec((_RB, 1), lambda i: (i, 0))
    return pl.pallas_call(
        _final_body,
        grid=(N // _RB,),
        in_specs=[full, full, full, col,
                  pl.BlockSpec((1, D), lambda i: (0, 0))],
        out_specs=full,
        out_shape=jax.ShapeDtypeStruct((N, D), jnp.float32),
    )(q0, q1, g2, dinv, b2r)


def kernel(x, edge_index, W1, b1, W2, b2):
    ei = edge_index.astype(jnp.int32)
    pad = EPAD - E
    src = jnp.concatenate([ei[0], jnp.zeros((pad,), jnp.int32)])
    dump = N + (jnp.arange(pad, dtype=jnp.int32) % (ACC_ROWS - N))
    dst = jnp.concatenate([ei[1], dump])
    src3 = src.reshape(NW, KCH, CH)
    dst3 = dst.reshape(NW, KCH, CH)

    hist = _sc_hist(dst3)
    hist4 = hist.reshape(NW, ACC_ROWS // D, D)
    dinv = _tc_dinv(hist4).reshape(ACC_ROWS, 1)[:N]

    g1 = _tc_mm_scale(x, W1, dinv)
    p = _sc_aggregate(g1, src3, dst3)
    g2 = _tc_layer2(p[0, :N], p[1, :N], g1, dinv, b1.reshape(1, D), W2)
    q = _sc_aggregate(g2, src3, dst3)
    return _tc_final(q[0, :N], q[1, :N], g2, dinv, b2.reshape(1, D))
